# Initial kernel scaffold; baseline (speedup 1.0000x reference)
import jax
import jax.numpy as jnp
from jax.experimental import pallas as pl


def kernel(node_feats, node_xy, node_adj_ids, edge_ids, Wn1, bn1, Wn2, bn2, We1, be1, We2, be2):
    raise NotImplementedError("write your pallas kernel here")


# speedup vs baseline: 2.5608x; 2.5608x over previous
import functools

import jax
import jax.numpy as jnp
from jax import lax
from jax.experimental import pallas as pl
from jax.experimental.pallas import tpu as pltpu
from jax.experimental.pallas import tpu_sc as plsc

_NW = 32
_B = 128
_L = 16
_H = 128


def _tc_tables(nf, Wn1, bn1, Wn2, bn2, We1a, We1b, be1):
    n = nf.shape[0]
    d = nf.shape[1]
    bn = 400
    assert n % bn == 0

    def body(nf_ref, wn1_ref, bn1_ref, wn2_ref, bn2_ref, we1a_ref, we1b_ref,
             be1_ref, ns_ref, t1_ref, t2_ref):
        x = nf_ref[...]
        h = jnp.maximum(
            jnp.dot(x, wn1_ref[...], preferred_element_type=jnp.float32)
            + bn1_ref[...], 0.0)
        ns_ref[...] = (jnp.sum(h * wn2_ref[...], axis=1, keepdims=True)
                       + bn2_ref[...])
        t1_ref[...] = (jnp.dot(x, we1a_ref[...],
                               preferred_element_type=jnp.float32)
                       + be1_ref[...])
        t2_ref[...] = jnp.dot(x, we1b_ref[...],
                              preferred_element_type=jnp.float32)

    return pl.pallas_call(
        body,
        grid=(n // bn,),
        in_specs=[
            pl.BlockSpec((bn, d), lambda i: (i, 0)),
            pl.BlockSpec((d, _H), lambda i: (0, 0)),
            pl.BlockSpec((1, _H), lambda i: (0, 0)),
            pl.BlockSpec((1, _H), lambda i: (0, 0)),
            pl.BlockSpec((1, 1), lambda i: (0, 0)),
            pl.BlockSpec((d, _H), lambda i: (0, 0)),
            pl.BlockSpec((d, _H), lambda i: (0, 0)),
            pl.BlockSpec((1, _H), lambda i: (0, 0)),
        ],
        out_specs=[
            pl.BlockSpec((bn, 1), lambda i: (i, 0)),
            pl.BlockSpec((bn, _H), lambda i: (i, 0)),
            pl.BlockSpec((bn, _H), lambda i: (i, 0)),
        ],
        out_shape=[
            jax.ShapeDtypeStruct((n, 1), jnp.float32),
            jax.ShapeDtypeStruct((n, _H), jnp.float32),
            jax.ShapeDtypeStruct((n, _H), jnp.float32),
        ],
    )(nf, Wn1, bn1, Wn2, bn2, We1a, We1b, be1)


def _sc_edge_partials(t1, t2, esrc, edst, w2):
    epad = esrc.shape[0]
    chunks_per_worker = epad // (_NW * _B)
    mesh = plsc.VectorSubcoreMesh(core_axis_name="c", subcore_axis_name="s")

    @functools.partial(
        pl.kernel,
        mesh=mesh,
        out_type=jax.ShapeDtypeStruct((epad * _L,), jnp.float32),
        scratch_types=[
            pltpu.VMEM((_B,), jnp.int32),
            pltpu.VMEM((_B,), jnp.int32),
            pltpu.VMEM((_B, _H), jnp.float32),
            pltpu.VMEM((_B, _H), jnp.float32),
            pltpu.VMEM((_B * _L,), jnp.float32),
            pltpu.VMEM((_H,), jnp.float32),
            pltpu.SemaphoreType.DMA,
            pltpu.SemaphoreType.DMA,
        ],
    )
    def k(t1_hbm, t2_hbm, esrc_hbm, edst_hbm, w2_hbm, out_hbm,
          sidx, didx, srows, drows, outv, w2v, sem1, sem2):
        wid = lax.axis_index("s") * 2 + lax.axis_index("c")
        pltpu.sync_copy(w2_hbm, w2v)
        w2c = [w2v[pl.ds(_L * j, _L)] for j in range(_H // _L)]

        def chunk_body(i, carry):
            base = (i * _NW + wid) * _B
            pltpu.sync_copy(esrc_hbm.at[pl.ds(base, _B)], sidx)
            pltpu.sync_copy(edst_hbm.at[pl.ds(base, _B)], didx)
            cp1 = pltpu.async_copy(t1_hbm.at[sidx], srows, sem1)
            cp2 = pltpu.async_copy(t2_hbm.at[didx], drows, sem2)
            cp1.wait()
            cp2.wait()

            def edge_body(e, c):
                parts = []
                for j in range(_H // _L):
                    sl = pl.ds(_L * j, _L)
                    u = jnp.maximum(srows[e, sl] + drows[e, sl], 0.0)
                    parts.append(u * w2c[j])
                while len(parts) > 1:
                    parts = [a + b for a, b in zip(parts[::2], parts[1::2])]
                outv[pl.ds(e * _L, _L)] = parts[0]
                return c

            lax.fori_loop(0, _B, edge_body, 0, unroll=4)
            pltpu.sync_copy(outv, out_hbm.at[pl.ds(base * _L, _B * _L)])
            return carry

        lax.fori_loop(0, chunks_per_worker, chunk_body, 0)

    return k(t1, t2, esrc, edst, w2)


def _tc_finalize(partials, seg, b2):
    r = partials.shape[0]
    br = 512
    assert r % br == 0

    def body(p_ref, s_ref, b2_ref, o_ref):
        o_ref[...] = (jnp.dot(p_ref[...], s_ref[...],
                              preferred_element_type=jnp.float32)
                      + b2_ref[...])

    return pl.pallas_call(
        body,
        grid=(r // br,),
        in_specs=[
            pl.BlockSpec((br, 128), lambda i: (i, 0)),
            pl.BlockSpec((128, 8), lambda i: (0, 0)),
            pl.BlockSpec((1, 1), lambda i: (0, 0)),
        ],
        out_specs=pl.BlockSpec((br, 8), lambda i: (i, 0)),
        out_shape=jax.ShapeDtypeStruct((r, 8), jnp.float32),
    )(partials, seg, b2)


def kernel(node_feats, node_xy, node_adj_ids, edge_ids, Wn1, bn1, Wn2, bn2,
           We1, be1, We2, be2):
    d = node_feats.shape[1]
    e = edge_ids.shape[1]

    node_scores, t1, t2 = _tc_tables(
        node_feats, Wn1, bn1.reshape(1, -1), Wn2.reshape(1, -1),
        bn2.reshape(1, 1), We1[:d], We1[d:], be1.reshape(1, -1))

    epad = -(-e // (_NW * _B)) * (_NW * _B)
    esrc = jnp.pad(edge_ids[0], (0, epad - e))
    edst = jnp.pad(edge_ids[1], (0, epad - e))
    w2 = We2.reshape(-1)

    partials = _sc_edge_partials(t1, t2, esrc, edst, w2)
    seg = (jnp.arange(128)[:, None] // _L
           == jnp.arange(8)[None, :]).astype(jnp.float32)
    sums = _tc_finalize(partials.reshape(epad * _L // 128, 128), seg,
                        be2.reshape(1, 1))
    edge_scores = sums.reshape(epad, 1)[:e]
    return (node_scores, edge_scores)
